# baseline (speedup 1.0000x reference)
import jax
import jax.numpy as jnp
from jax.experimental import pallas as pl
from jax.experimental.pallas import tpu as pltpu

D_MODEL = 768
D_FF = 4 * D_MODEL
TOKEN_BLOCK = 1024
F_CHUNK = 768
N_CHUNKS = D_FF // F_CHUNK

_NT = (((1,), (1,)), ((), ()))


def _moe_kernel(x_ref, gate_ref, w1_ref, b1_ref, w2_ref, w3_ref,
                b2_ref, b3_ref, out_ref,
                v_ref, rw0_ref, rw1_ref, s0_ref, s1_ref):
    i = pl.program_id(0)
    kf = pl.program_id(1)
    lo = kf * F_CHUNK

    @pl.when(i == 0)
    def _compute_v_chunk():
        v_ref[:, :, pl.ds(lo, F_CHUNK)] = jax.lax.dot_general(
            w3_ref[...], w2_ref[...],
            dimension_numbers=(((2,), (1,)), ((0,), (0,))),
            preferred_element_type=jnp.float32,
        )

    x = x_ref[...]

    @pl.when(kf == 0)
    def _routing_and_init():
        logits = jax.lax.dot_general(x, gate_ref[...], _NT,
                                     preferred_element_type=jnp.float32)
        m1 = jnp.max(logits, axis=1, keepdims=True)
        iota = jax.lax.broadcasted_iota(jnp.int32, logits.shape, 1)
        first_max = jnp.min(jnp.where(logits == m1, iota, logits.shape[1]),
                            axis=1, keepdims=True)
        masked = jnp.where(iota == first_max, -jnp.inf, logits)
        m2 = jnp.max(masked, axis=1, keepdims=True)
        rw0_ref[...] = 1.0 / (1.0 + jnp.exp(m2 - m1))
        rw1_ref[...] = 1.0 - rw0_ref[...]

        w3m = w3_ref[:, 0, :]
        c = jnp.sum(b2_ref[:, 0, :] * w3m, axis=1, keepdims=True) \
            + b3_ref[:, 0, :]
        xw3 = jax.lax.dot_general(x, w3m, _NT,
                                  preferred_element_type=jnp.float32)
        s0_ref[...] = xw3[:, 0:1] + c[0:1, 0:1]
        s1_ref[...] = xw3[:, 1:2] + c[1:2, 0:1]

    for j, s_ref in ((0, s0_ref), (1, s1_ref)):
        h = jax.lax.dot_general(x, w1_ref[j], _NT,
                                preferred_element_type=jnp.float32)
        h = jnp.maximum(h + b1_ref[j, :, pl.ds(lo, F_CHUNK)], 0.0)
        s_ref[...] = s_ref[...] + jnp.sum(h * v_ref[j, :, pl.ds(lo, F_CHUNK)],
                                          axis=1, keepdims=True)

    @pl.when(kf == N_CHUNKS - 1)
    def _combine():
        out_ref[...] = rw0_ref[...] * s0_ref[...] + rw1_ref[...] * s1_ref[...]


def kernel(hidden_states, gate_w, W1, b1, W2, b2, W3, b3):
    n, d = hidden_states.shape
    f = D_FF
    e = gate_w.shape[0]
    b = TOKEN_BLOCK
    nk = N_CHUNKS

    nb = n // b
    out = pl.pallas_call(
        _moe_kernel,
        grid=(nb, nk),
        in_specs=[
            pl.BlockSpec((b, d), lambda i, kf: (i, 0)),
            pl.BlockSpec((e, d), lambda i, kf: (0, 0)),
            pl.BlockSpec((2, F_CHUNK, d), lambda i, kf: (0, kf, 0)),
            pl.BlockSpec((2, 1, f), lambda i, kf: (0, 0, 0)),
            pl.BlockSpec((2, d, F_CHUNK),
                         lambda i, kf: (0, 0, jnp.where(i < 1, kf, nk - 1))),
            pl.BlockSpec((2, 1, d), lambda i, kf: (0, 0, 0)),
            pl.BlockSpec((2, 1, d), lambda i, kf: (0, 0, 0)),
            pl.BlockSpec((2, 1, 1), lambda i, kf: (0, 0, 0)),
        ],
        out_specs=pl.BlockSpec((b, 1), lambda i, kf: (i, 0)),
        out_shape=jax.ShapeDtypeStruct((n, 1), jnp.float32),
        scratch_shapes=[
            pltpu.VMEM((2, 1, f), jnp.float32),
            pltpu.VMEM((b, 1), jnp.float32),
            pltpu.VMEM((b, 1), jnp.float32),
            pltpu.VMEM((b, 1), jnp.float32),
            pltpu.VMEM((b, 1), jnp.float32),
        ],
    )(hidden_states, gate_w, W1, b1.reshape(e, 1, f), W2,
      W3, b2.reshape(e, 1, d), b3.reshape(e, 1, 1))
    return out


# speedup vs baseline: 1.1799x; 1.1799x over previous
import jax
import jax.numpy as jnp
from jax.experimental import pallas as pl
from jax.experimental.pallas import tpu as pltpu

D_MODEL = 768
D_FF = 4 * D_MODEL
TOKEN_BLOCK = 1024

_NT = (((1,), (1,)), ((), ()))


def _moe_kernel(x_ref, gate_ref, w1_ref, b1_ref, w2_ref, w3_ref,
                b2_ref, b3_ref, out_ref, v_ref):
    i = pl.program_id(0)

    @pl.when(i == 0)
    def _compute_v():
        v_ref[...] = jax.lax.dot_general(
            w3_ref[...], w2_ref[...],
            dimension_numbers=(((2,), (1,)), ((0,), (0,))),
            preferred_element_type=jnp.float32,
        )

    x = x_ref[...]

    logits = jax.lax.dot_general(x, gate_ref[...], _NT,
                                 preferred_element_type=jnp.float32)
    m1 = jnp.max(logits, axis=1, keepdims=True)
    iota = jax.lax.broadcasted_iota(jnp.int32, logits.shape, 1)
    first_max = jnp.min(jnp.where(logits == m1, iota, logits.shape[1]),
                        axis=1, keepdims=True)
    masked = jnp.where(iota == first_max, -jnp.inf, logits)
    m2 = jnp.max(masked, axis=1, keepdims=True)
    rw0 = 1.0 / (1.0 + jnp.exp(m2 - m1))
    rw1 = 1.0 - rw0

    w3m = w3_ref[:, 0, :]
    c = jnp.sum(b2_ref[:, 0, :] * w3m, axis=1, keepdims=True) \
        + b3_ref[:, 0, :]

    xw3 = jax.lax.dot_general(x, w3m, _NT,
                              preferred_element_type=jnp.float32)

    w1all = w1_ref[...].reshape(2 * D_FF, D_MODEL)
    h = jax.lax.dot_general(x, w1all, _NT,
                            preferred_element_type=jnp.float32)
    h = jnp.maximum(h + b1_ref[...].reshape(1, 2 * D_FF), 0.0)
    vall = v_ref[...].reshape(1, 2 * D_FF)
    hv = h * vall
    s0 = jnp.sum(hv[:, :D_FF], axis=1, keepdims=True) \
        + xw3[:, 0:1] + c[0:1, 0:1]
    s1 = jnp.sum(hv[:, D_FF:], axis=1, keepdims=True) \
        + xw3[:, 1:2] + c[1:2, 0:1]
    out_ref[...] = rw0 * s0 + rw1 * s1


def kernel(hidden_states, gate_w, W1, b1, W2, b2, W3, b3):
    n, d = hidden_states.shape
    f = D_FF
    e = gate_w.shape[0]

    nb = n // TOKEN_BLOCK
    out = pl.pallas_call(
        _moe_kernel,
        grid=(nb,),
        in_specs=[
            pl.BlockSpec((TOKEN_BLOCK, d), lambda i: (i, 0)),
            pl.BlockSpec((e, d), lambda i: (0, 0)),
            pl.BlockSpec((2, f, d), lambda i: (0, 0, 0)),
            pl.BlockSpec((2, 1, f), lambda i: (0, 0, 0)),
            pl.BlockSpec((2, d, f), lambda i: (0, 0, 0)),
            pl.BlockSpec((2, 1, d), lambda i: (0, 0, 0)),
            pl.BlockSpec((2, 1, d), lambda i: (0, 0, 0)),
            pl.BlockSpec((2, 1, 1), lambda i: (0, 0, 0)),
        ],
        out_specs=pl.BlockSpec((TOKEN_BLOCK, 1), lambda i: (i, 0)),
        out_shape=jax.ShapeDtypeStruct((n, 1), jnp.float32),
        scratch_shapes=[pltpu.VMEM((2, 1, f), jnp.float32)],
    )(hidden_states, gate_w, W1, b1.reshape(e, 1, f), W2,
      W3, b2.reshape(e, 1, d), b3.reshape(e, 1, 1))
    return out
